# Initial kernel scaffold; baseline (speedup 1.0000x reference)
import jax
import jax.numpy as jnp
from jax.experimental import pallas as pl


def kernel(arg0_1, arg3_1, convert_element_type, convert_element_type_1):
    raise NotImplementedError("write your pallas kernel here")


# speedup vs baseline: 37.9097x; 37.9097x over previous
import functools

import jax
import jax.numpy as jnp
from jax import lax
from jax.experimental import pallas as pl
from jax.experimental.pallas import tpu as pltpu
from jax.experimental.pallas import tpu_sc as plsc

_N_EDGES = 6400000
_N_NODES = 100000
_NC = 2
_NS = 16
_NT = _NC * _NS
_EPT = _N_EDGES // _NT
_CHUNK = 8000
_NCHUNK = _EPT // _CHUNK
_VPC = _CHUNK // 16
_ACC = 100352
_SENT = _N_NODES


def _sc_partial_counts(edges):
    mesh = plsc.VectorSubcoreMesh(core_axis_name="c", subcore_axis_name="s")

    @functools.partial(
        pl.kernel,
        out_type=jax.ShapeDtypeStruct((_NT, _ACC), jnp.float32),
        mesh=mesh,
        scratch_types=[
            pltpu.VMEM((_ACC,), jnp.float32),
            pltpu.VMEM((_CHUNK + 16,), jnp.int32),
        ],
        compiler_params=pltpu.CompilerParams(needs_layout_passes=False),
    )
    def body(edges_hbm, part_hbm, acc, ebuf):
        cid = lax.axis_index("c")
        sid = lax.axis_index("s")
        wid = cid * _NS + sid
        base = wid * _EPT

        zero16 = jnp.zeros((16,), jnp.float32)

        def zbody(i, _):
            acc[pl.ds(i * 16, 16)] = zero16
            return 0

        lax.fori_loop(0, _ACC // 16, zbody, 0)

        iota = lax.iota(jnp.int32, 16)
        sentv = jnp.full((16,), _SENT, jnp.int32)

        def chunk_body(g, _):
            off = pl.multiple_of(base + g * _CHUNK, 8)
            is_last = jnp.logical_and(wid == _NT - 1, g == _NCHUNK - 1)

            @pl.when(jnp.logical_not(is_last))
            def _():
                pltpu.sync_copy(edges_hbm.at[pl.ds(off, _CHUNK + 16)], ebuf)

            @pl.when(is_last)
            def _():
                pltpu.sync_copy(
                    edges_hbm.at[pl.ds(off, _CHUNK)], ebuf.at[pl.ds(0, _CHUNK)]
                )
                ebuf[pl.ds(_CHUNK, 16)] = sentv

            def vbody(k, _):
                cur = ebuf[pl.ds(k * 16, 16)]
                nxt = ebuf[pl.ds(k * 16 + 1, 16)]
                m = cur != nxt
                pos = (iota + (off + k * 16 + 1)).astype(jnp.float32)
                plsc.addupdate_scatter(acc, [cur], pos, mask=m)
                plsc.addupdate_scatter(acc, [nxt], -pos, mask=m)
                return 0

            lax.fori_loop(0, _VPC, vbody, 0)
            return 0

        lax.fori_loop(0, _NCHUNK, chunk_body, 0)
        pltpu.sync_copy(acc, part_hbm.at[wid])

    return body(edges)


def _combine(part, a3, cet):
    rows = _ACC // 128
    block_rows = 112

    def body(p_ref, a_ref, c_ref, o_ref):
        s = jnp.sum(p_ref[...], axis=0)
        o_ref[...] = a_ref[...] / (c_ref[...] + s)

    return pl.pallas_call(
        body,
        grid=(rows // block_rows,),
        in_specs=[
            pl.BlockSpec((_NT, block_rows, 128), lambda i: (0, i, 0)),
            pl.BlockSpec((block_rows, 128), lambda i: (i, 0)),
            pl.BlockSpec((block_rows, 128), lambda i: (i, 0)),
        ],
        out_specs=pl.BlockSpec((block_rows, 128), lambda i: (i, 0)),
        out_shape=jax.ShapeDtypeStruct((rows, 128), jnp.float32),
    )(part, a3, cet)


def kernel(arg0_1, arg3_1, convert_element_type, convert_element_type_1):
    del convert_element_type_1
    edges = arg0_1.astype(jnp.int32)
    part = _sc_partial_counts(edges)
    rows = _ACC // 128
    a3 = jnp.pad(arg3_1, (0, _ACC - _N_NODES)).reshape(rows, 128)
    cet = jnp.pad(convert_element_type, (0, _ACC - _N_NODES)).reshape(rows, 128)
    out = _combine(part.reshape(_NT, rows, 128), a3, cet)
    return out.reshape(_ACC)[:_N_NODES]


# speedup vs baseline: 47.2560x; 1.2465x over previous
import functools

import jax
import jax.numpy as jnp
from jax import lax
from jax.experimental import pallas as pl
from jax.experimental.pallas import tpu as pltpu
from jax.experimental.pallas import tpu_sc as plsc

_N_EDGES = 6400000
_N_NODES = 100000
_NC = 2
_NS = 16
_NT = _NC * _NS
_EPT = _N_EDGES // _NT
_CHUNK = 8000
_NCHUNK = _EPT // _CHUNK
_VPC = _CHUNK // 16
_ACC = 100352
_SENT = _N_NODES


def _sc_partial_counts(edges):
    mesh = plsc.VectorSubcoreMesh(core_axis_name="c", subcore_axis_name="s")

    @functools.partial(
        pl.kernel,
        out_type=jax.ShapeDtypeStruct((_NT, _ACC), jnp.float32),
        mesh=mesh,
        scratch_types=[
            pltpu.VMEM((_ACC,), jnp.float32),
            pltpu.VMEM((_CHUNK + 16,), jnp.int32),
            pltpu.VMEM((_CHUNK + 16,), jnp.int32),
            pltpu.SemaphoreType.DMA,
            pltpu.SemaphoreType.DMA,
        ],
        compiler_params=pltpu.CompilerParams(needs_layout_passes=False),
    )
    def body(edges_hbm, part_hbm, acc, ebuf0, ebuf1, sem0, sem1):
        cid = lax.axis_index("c")
        sid = lax.axis_index("s")
        wid = cid * _NS + sid
        base = wid * _EPT
        bufs = (ebuf0, ebuf1)
        sems = (sem0, sem1)

        zero16 = jnp.zeros((16,), jnp.float32)

        def zbody(i, _):
            for u in range(16):
                acc[pl.ds(i * 256 + u * 16, 16)] = zero16
            return 0

        lax.fori_loop(0, _ACC // 256, zbody, 0)

        iota1 = lax.iota(jnp.int32, 16) + 1
        sentv = jnp.full((16,), _SENT, jnp.int32)
        is_tail_tile = wid == _NT - 1

        def dma_descs(g, eb, sem):
            off = pl.multiple_of(base + g * _CHUNK, 8)
            full = pltpu.make_async_copy(
                edges_hbm.at[pl.ds(off, _CHUNK + 16)], eb, sem
            )
            short = pltpu.make_async_copy(
                edges_hbm.at[pl.ds(off, _CHUNK)], eb.at[pl.ds(0, _CHUNK)], sem
            )
            return full, short

        def dma_start(g):
            full, short = dma_descs(g, bufs[g % 2], sems[g % 2])
            if g == _NCHUNK - 1:
                @pl.when(is_tail_tile)
                def _():
                    short.start()

                @pl.when(jnp.logical_not(is_tail_tile))
                def _():
                    full.start()
            else:
                full.start()

        def dma_wait(g):
            eb = bufs[g % 2]
            full, short = dma_descs(g, eb, sems[g % 2])
            if g == _NCHUNK - 1:
                @pl.when(is_tail_tile)
                def _():
                    short.wait()
                    eb[pl.ds(_CHUNK, 16)] = sentv

                @pl.when(jnp.logical_not(is_tail_tile))
                def _():
                    full.wait()
            else:
                full.wait()

        _U = 10
        dma_start(0)
        for g in range(_NCHUNK):
            if g + 1 < _NCHUNK:
                dma_start(g + 1)
            dma_wait(g)
            eb = bufs[g % 2]
            off = base + g * _CHUNK

            def vbody(k, _, eb=eb, off=off):
                kb = k * (_U * 16)
                for u in range(_U):
                    idx = kb + u * 16
                    cur = eb[pl.ds(idx, 16)]
                    nxt = eb[pl.ds(idx + 1, 16)]
                    m = cur != nxt
                    pos = (iota1 + (off + idx)).astype(jnp.float32)
                    plsc.addupdate_scatter(acc, [cur], pos, mask=m)
                    plsc.addupdate_scatter(acc, [nxt], -pos, mask=m)
                return 0

            lax.fori_loop(0, _VPC // _U, vbody, 0)

        pltpu.sync_copy(acc, part_hbm.at[wid])

    return body(edges)


def _combine(part, a3, cet):
    rows = _ACC // 128
    block_rows = 112

    def body(p_ref, a_ref, c_ref, o_ref):
        s = jnp.sum(p_ref[...], axis=0)
        o_ref[...] = a_ref[...] / (c_ref[...] + s)

    return pl.pallas_call(
        body,
        grid=(rows // block_rows,),
        in_specs=[
            pl.BlockSpec((_NT, block_rows, 128), lambda i: (0, i, 0)),
            pl.BlockSpec((block_rows, 128), lambda i: (i, 0)),
            pl.BlockSpec((block_rows, 128), lambda i: (i, 0)),
        ],
        out_specs=pl.BlockSpec((block_rows, 128), lambda i: (i, 0)),
        out_shape=jax.ShapeDtypeStruct((rows, 128), jnp.float32),
    )(part, a3, cet)


def kernel(arg0_1, arg3_1, convert_element_type, convert_element_type_1):
    del convert_element_type_1
    edges = arg0_1.astype(jnp.int32)
    part = _sc_partial_counts(edges)
    rows = _ACC // 128
    a3 = jnp.pad(arg3_1, (0, _ACC - _N_NODES)).reshape(rows, 128)
    cet = jnp.pad(convert_element_type, (0, _ACC - _N_NODES)).reshape(rows, 128)
    out = _combine(part.reshape(_NT, rows, 128), a3, cet)
    return out.reshape(_ACC)[:_N_NODES]


# speedup vs baseline: 107.8204x; 2.2816x over previous
import functools

import jax
import jax.numpy as jnp
from jax import lax
from jax.experimental import pallas as pl
from jax.experimental.pallas import tpu as pltpu
from jax.experimental.pallas import tpu_sc as plsc

_N_EDGES = 6400000
_N_NODES = 100000
_NC = 2
_NS = 16
_NT = _NC * _NS
_EPT = _N_EDGES // _NT
_CHUNK = 8000
_NCHUNK = _EPT // _CHUNK
_VPC = _CHUNK // 16
_ACC = 100352
_SENT = _N_NODES


def _sc_partial_counts(edges):
    mesh = plsc.VectorSubcoreMesh(core_axis_name="c", subcore_axis_name="s")

    @functools.partial(
        pl.kernel,
        out_type=jax.ShapeDtypeStruct((_NT, _ACC), jnp.float32),
        mesh=mesh,
        scratch_types=[
            pltpu.VMEM((_ACC,), jnp.float32),
            pltpu.VMEM((_CHUNK + 16,), jnp.int32),
            pltpu.VMEM((_CHUNK + 16,), jnp.int32),
            pltpu.SemaphoreType.DMA,
            pltpu.SemaphoreType.DMA,
        ],
        compiler_params=pltpu.CompilerParams(needs_layout_passes=False),
    )
    def body(edges_hbm, part_hbm, acc, ebuf0, ebuf1, sem0, sem1):
        cid = lax.axis_index("c")
        sid = lax.axis_index("s")
        wid = cid * _NS + sid
        base = wid * _EPT
        bufs = (ebuf0, ebuf1)
        sems = (sem0, sem1)

        zero16 = jnp.zeros((16,), jnp.float32)

        def zbody(i, _):
            for u in range(16):
                acc[pl.ds(i * 256 + u * 16, 16)] = zero16
            return 0

        lax.fori_loop(0, _ACC // 256, zbody, 0)

        iota1 = lax.iota(jnp.int32, 16) + 1
        sentv = jnp.full((16,), _SENT, jnp.int32)
        is_tail_tile = wid == _NT - 1

        def dma_descs(g, eb, sem):
            off = pl.multiple_of(base + g * _CHUNK, 8)
            full = pltpu.make_async_copy(
                edges_hbm.at[pl.ds(off, _CHUNK + 16)], eb, sem
            )
            short = pltpu.make_async_copy(
                edges_hbm.at[pl.ds(off, _CHUNK)], eb.at[pl.ds(0, _CHUNK)], sem
            )
            return full, short

        def dma_start(g):
            full, short = dma_descs(g, bufs[g % 2], sems[g % 2])
            if g == _NCHUNK - 1:
                @pl.when(is_tail_tile)
                def _():
                    short.start()

                @pl.when(jnp.logical_not(is_tail_tile))
                def _():
                    full.start()
            else:
                full.start()

        def dma_wait(g):
            eb = bufs[g % 2]
            full, short = dma_descs(g, eb, sems[g % 2])
            if g == _NCHUNK - 1:
                @pl.when(is_tail_tile)
                def _():
                    short.wait()
                    eb[pl.ds(_CHUNK, 16)] = sentv

                @pl.when(jnp.logical_not(is_tail_tile))
                def _():
                    full.wait()
            else:
                full.wait()

        _U = 10
        dma_start(0)
        for g in range(_NCHUNK):
            if g + 1 < _NCHUNK:
                dma_start(g + 1)
            dma_wait(g)
            eb = bufs[g % 2]
            off = base + g * _CHUNK

            @plsc.parallel_loop(0, _VPC, unroll=_U)
            def _(k, eb=eb, off=off):
                idx = k * 16
                cur = eb[pl.ds(idx, 16)]
                nxt = eb[pl.ds(idx + 1, 16)]
                m = cur != nxt
                pos = (iota1 + (off + idx)).astype(jnp.float32)
                plsc.addupdate_scatter(acc, [cur], pos, mask=m)
                plsc.addupdate_scatter(acc, [nxt], -pos, mask=m)

        pltpu.sync_copy(acc, part_hbm.at[wid])

    return body(edges)


_CB = 12544


def _combine(part, a3, cet):
    def body(p_ref, a_ref, c_ref, o_ref):
        s = jnp.sum(p_ref[...], axis=0)
        o_ref[...] = a_ref[...] / (c_ref[...] + s)

    return pl.pallas_call(
        body,
        out_shape=jax.ShapeDtypeStruct((_ACC,), jnp.float32),
    )(part, a3, cet)


def kernel(arg0_1, arg3_1, convert_element_type, convert_element_type_1):
    del convert_element_type_1
    edges = arg0_1.astype(jnp.int32)
    part = _sc_partial_counts(edges)
    a3 = jnp.pad(arg3_1, (0, _ACC - _N_NODES))
    cet = jnp.pad(convert_element_type, (0, _ACC - _N_NODES))
    out = _combine(part, a3, cet)
    return out[:_N_NODES]


# speedup vs baseline: 109.4622x; 1.0152x over previous
import functools

import jax
import jax.numpy as jnp
from jax import lax
from jax.experimental import pallas as pl
from jax.experimental.pallas import tpu as pltpu
from jax.experimental.pallas import tpu_sc as plsc

_N_EDGES = 6400000
_N_NODES = 100000
_NC = 2
_NS = 16
_NT = _NC * _NS
_EPT = _N_EDGES // _NT
_CHUNK = 10000
_NCHUNK = _EPT // _CHUNK
_VPC = _CHUNK // 16
_ACC = 100352
_SENT = _N_NODES


def _sc_partial_counts(edges):
    mesh = plsc.VectorSubcoreMesh(core_axis_name="c", subcore_axis_name="s")

    @functools.partial(
        pl.kernel,
        out_type=jax.ShapeDtypeStruct((_NT, _ACC), jnp.float32),
        mesh=mesh,
        scratch_types=[
            pltpu.VMEM((_ACC,), jnp.float32),
            pltpu.VMEM((_CHUNK + 16,), jnp.int32),
            pltpu.VMEM((_CHUNK + 16,), jnp.int32),
            pltpu.SemaphoreType.DMA,
            pltpu.SemaphoreType.DMA,
        ],
        compiler_params=pltpu.CompilerParams(needs_layout_passes=False),
    )
    def body(edges_hbm, part_hbm, acc, ebuf0, ebuf1, sem0, sem1):
        cid = lax.axis_index("c")
        sid = lax.axis_index("s")
        wid = cid * _NS + sid
        base = wid * _EPT
        bufs = (ebuf0, ebuf1)
        sems = (sem0, sem1)

        zero16 = jnp.zeros((16,), jnp.float32)

        iota1 = lax.iota(jnp.int32, 16) + 1
        sentv = jnp.full((16,), _SENT, jnp.int32)
        is_tail_tile = wid == _NT - 1

        def dma_descs(g, eb, sem):
            off = pl.multiple_of(base + g * _CHUNK, 8)
            full = pltpu.make_async_copy(
                edges_hbm.at[pl.ds(off, _CHUNK + 16)], eb, sem
            )
            short = pltpu.make_async_copy(
                edges_hbm.at[pl.ds(off, _CHUNK)], eb.at[pl.ds(0, _CHUNK)], sem
            )
            return full, short

        def dma_start(g):
            full, short = dma_descs(g, bufs[g % 2], sems[g % 2])
            if g == _NCHUNK - 1:
                @pl.when(is_tail_tile)
                def _():
                    short.start()

                @pl.when(jnp.logical_not(is_tail_tile))
                def _():
                    full.start()
            else:
                full.start()

        def dma_wait(g):
            eb = bufs[g % 2]
            full, short = dma_descs(g, eb, sems[g % 2])
            if g == _NCHUNK - 1:
                @pl.when(is_tail_tile)
                def _():
                    short.wait()
                    eb[pl.ds(_CHUNK, 16)] = sentv

                @pl.when(jnp.logical_not(is_tail_tile))
                def _():
                    full.wait()
            else:
                full.wait()

        _U = 10
        dma_start(0)

        def zbody(i, _):
            for u in range(16):
                acc[pl.ds(i * 256 + u * 16, 16)] = zero16
            return 0

        lax.fori_loop(0, _ACC // 256, zbody, 0)

        for g in range(_NCHUNK):
            if g + 1 < _NCHUNK:
                dma_start(g + 1)
            dma_wait(g)
            eb = bufs[g % 2]
            off = base + g * _CHUNK

            @plsc.parallel_loop(0, _VPC, unroll=_U)
            def _(k, eb=eb, off=off):
                idx = k * 16
                cur = eb[pl.ds(idx, 16)]
                nxt = eb[pl.ds(idx + 1, 16)]
                m = cur != nxt
                pos = (iota1 + (off + idx)).astype(jnp.float32)
                plsc.addupdate_scatter(acc, [cur], pos, mask=m)
                plsc.addupdate_scatter(acc, [nxt], -pos, mask=m)

        pltpu.sync_copy(acc, part_hbm.at[wid])

    return body(edges)


_CB = 12544


def _combine(part, a3, cet):
    def body(p_ref, a_ref, c_ref, o_ref):
        s = jnp.sum(p_ref[...], axis=0)
        o_ref[...] = a_ref[...] / (c_ref[...] + s)

    return pl.pallas_call(
        body,
        out_shape=jax.ShapeDtypeStruct((_ACC,), jnp.float32),
    )(part, a3, cet)


def kernel(arg0_1, arg3_1, convert_element_type, convert_element_type_1):
    del convert_element_type_1
    edges = arg0_1.astype(jnp.int32)
    part = _sc_partial_counts(edges)
    a3 = jnp.pad(arg3_1, (0, _ACC - _N_NODES))
    cet = jnp.pad(convert_element_type, (0, _ACC - _N_NODES))
    out = _combine(part, a3, cet)
    return out[:_N_NODES]
